# Initial kernel scaffold; baseline (speedup 1.0000x reference)
import jax
import jax.numpy as jnp
from jax.experimental import pallas as pl


def kernel(raw_cls_bbox, raw_prob):
    raise NotImplementedError("write your pallas kernel here")


# speedup vs baseline: 10.9279x; 10.9279x over previous
import functools

import jax
import jax.numpy as jnp
from jax import lax
from jax.experimental import pallas as pl
from jax.experimental.pallas import tpu as pltpu
from jax.experimental.pallas import tpu_sc as plsc

_NMS_T = 0.3
_SCORE_T = 0.05
_LANES = 16
_UNROLL = 4
_STEP = _LANES * _UNROLL


def _make_nms_kernel(n_cls, n_pad, n_real):
    nblk = n_pad // _LANES
    nstep = n_pad // _STEP

    def body(x1_h, y1_h, x2_h, y2_h, s_h,
             ox1_h, oy1_h, ox2_h, oy2_h, os_h, okeep_h,
             x1_v, y1_v, x2_v, y2_v, s_v, area_v, keep_v):
        wid = lax.axis_index("s") * 2 + lax.axis_index("c")

        @pl.when(wid < n_cls)
        def _work():
            pltpu.sync_copy(x1_h.at[wid], x1_v)
            pltpu.sync_copy(y1_h.at[wid], y1_v)
            pltpu.sync_copy(x2_h.at[wid], x2_v)
            pltpu.sync_copy(y2_h.at[wid], y2_v)
            pltpu.sync_copy(s_h.at[wid], s_v)

            def init_blk(b, carry):
                sl = pl.ds(b * _LANES, _LANES)
                w = jnp.maximum(x2_v[sl] - x1_v[sl], 0.0)
                h = jnp.maximum(y2_v[sl] - y1_v[sl], 0.0)
                area_v[sl] = w * h
                keep_v[sl] = jnp.where(s_v[sl] > -jnp.inf, 1.0, 0.0)
                return carry

            lax.fori_loop(0, nblk, init_blk, 0)

            def outer(i, carry):
                sli = pl.ds(i, _LANES)
                ki = keep_v[sli][0]

                @pl.when(ki > 0.0)
                def _suppress():
                    xi1 = x1_v[sli][0]
                    yi1 = y1_v[sli][0]
                    xi2 = x2_v[sli][0]
                    yi2 = y2_v[sli][0]
                    ai = area_v[sli][0]
                    g0 = (i + 1) // _STEP

                    def inner(g, c2):
                        base = g * _STEP
                        for u in range(_UNROLL):
                            sl = pl.ds(base + u * _LANES, _LANES)
                            xx1 = jnp.maximum(x1_v[sl], xi1)
                            yy1 = jnp.maximum(y1_v[sl], yi1)
                            xx2 = jnp.minimum(x2_v[sl], xi2)
                            yy2 = jnp.minimum(y2_v[sl], yi2)
                            inter = (jnp.maximum(xx2 - xx1, 0.0)
                                     * jnp.maximum(yy2 - yy1, 0.0))
                            union = (ai + area_v[sl]) - inter
                            iou = inter / jnp.maximum(union, 1e-9)
                            idx = (lax.iota(jnp.int32, _LANES)
                                   + (base + u * _LANES))
                            sup = (iou > _NMS_T) & (idx > i)
                            keep_v[sl] = jnp.where(sup, 0.0, keep_v[sl])
                        return c2

                    lax.fori_loop(g0, nstep, inner, 0)

                return carry

            lax.fori_loop(0, n_real, outer, 0)

            def mask_blk(b, carry):
                sl = pl.ds(b * _LANES, _LANES)
                k = keep_v[sl] > 0.0
                x1_v[sl] = jnp.where(k, x1_v[sl], 0.0)
                y1_v[sl] = jnp.where(k, y1_v[sl], 0.0)
                x2_v[sl] = jnp.where(k, x2_v[sl], 0.0)
                y2_v[sl] = jnp.where(k, y2_v[sl], 0.0)
                s_v[sl] = jnp.where(k, s_v[sl], 0.0)
                return carry

            lax.fori_loop(0, nblk, mask_blk, 0)

            pltpu.sync_copy(x1_v, ox1_h.at[wid])
            pltpu.sync_copy(y1_v, oy1_h.at[wid])
            pltpu.sync_copy(x2_v, ox2_h.at[wid])
            pltpu.sync_copy(y2_v, oy2_h.at[wid])
            pltpu.sync_copy(s_v, os_h.at[wid])
            pltpu.sync_copy(keep_v, okeep_h.at[wid])

    mesh = plsc.VectorSubcoreMesh(core_axis_name="c", subcore_axis_name="s")
    out_t = [jax.ShapeDtypeStruct((n_cls, n_pad), jnp.float32)] * 6
    scratch = [pltpu.VMEM((n_pad,), jnp.float32) for _ in range(7)]
    return pl.kernel(body, out_type=out_t, mesh=mesh, scratch_types=scratch)


def kernel(raw_cls_bbox, raw_prob):
    n, ncls = raw_prob.shape
    L = ncls - 1
    n_pad = -(-n // _STEP) * _STEP

    cls_bbox = raw_cls_bbox.reshape(n, ncls, 4)
    boxes_pc = jnp.transpose(cls_bbox[:, 1:, :], (1, 0, 2))
    probs_pc = jnp.transpose(raw_prob[:, 1:], (1, 0))
    scores = jnp.where(probs_pc > _SCORE_T, probs_pc, -jnp.inf)
    order = jnp.argsort(-scores, axis=1)
    b = jnp.take_along_axis(boxes_pc, order[:, :, None], axis=1)
    s = jnp.take_along_axis(scores, order, axis=1)

    pad = n_pad - n
    x1 = jnp.pad(b[:, :, 0], ((0, 0), (0, pad)))
    y1 = jnp.pad(b[:, :, 1], ((0, 0), (0, pad)))
    x2 = jnp.pad(b[:, :, 2], ((0, 0), (0, pad)))
    y2 = jnp.pad(b[:, :, 3], ((0, 0), (0, pad)))
    sp = jnp.pad(s, ((0, 0), (0, pad)), constant_values=-jnp.inf)

    fn = _make_nms_kernel(L, n_pad, n)
    ox1, oy1, ox2, oy2, os_, okeep = fn(x1, y1, x2, y2, sp)

    bbox = jnp.stack(
        [ox1[:, :n], oy1[:, :n], ox2[:, :n], oy2[:, :n]], axis=-1
    ).reshape(L * n, 4)
    score = os_[:, :n].reshape(L * n)
    keep = (okeep[:, :n] > 0.0).reshape(L * n)
    label = jnp.repeat(jnp.arange(L, dtype=jnp.int32), n)
    return bbox, label, score, keep


# speedup vs baseline: 46.7041x; 4.2739x over previous
import functools

import jax
import jax.numpy as jnp
from jax import lax
from jax.experimental import pallas as pl
from jax.experimental.pallas import tpu as pltpu
from jax.experimental.pallas import tpu_sc as plsc

_NMS_T = 0.3
_SCORE_T = 0.05
_LANES = 16
_UNROLL = 4
_STEP = _LANES * _UNROLL


def _make_nms_kernel(n_cls, n_pad, n_real):
    nblk = n_pad // _LANES
    nstep = n_pad // _STEP

    def body(x1_h, y1_h, x2_h, y2_h, s_h,
             ox1_h, oy1_h, ox2_h, oy2_h, os_h, okeep_h,
             x1_v, y1_v, x2_v, y2_v, s_v, area_v, keep_v):
        wid = lax.axis_index("s") * 2 + lax.axis_index("c")

        @pl.when(wid < n_cls)
        def _work():
            pltpu.sync_copy(x1_h.at[wid], x1_v)
            pltpu.sync_copy(y1_h.at[wid], y1_v)
            pltpu.sync_copy(x2_h.at[wid], x2_v)
            pltpu.sync_copy(y2_h.at[wid], y2_v)
            pltpu.sync_copy(s_h.at[wid], s_v)

            @plsc.parallel_loop(0, nblk, unroll=4)
            def _init_blk(b):
                sl = pl.ds(b * _LANES, _LANES)
                w = jnp.maximum(x2_v[sl] - x1_v[sl], 0.0)
                h = jnp.maximum(y2_v[sl] - y1_v[sl], 0.0)
                area_v[sl] = w * h
                keep_v[sl] = jnp.where(s_v[sl] > -jnp.inf, 1.0, 0.0)

            zeros16 = jnp.zeros((_LANES,), jnp.float32)

            def outer(i, carry):
                sli = pl.ds(i, _LANES)
                ki = keep_v[sli][0]

                @pl.when(ki > 0.0)
                def _suppress():
                    xi1 = x1_v[sli][0]
                    yi1 = y1_v[sli][0]
                    xi2 = x2_v[sli][0]
                    yi2 = y2_v[sli][0]
                    ai = area_v[sli][0]
                    g0 = (i + 1) // _LANES

                    @plsc.parallel_loop(g0, nblk, unroll=4)
                    def _sweep(g):
                        base = g * _LANES
                        sl = pl.ds(base, _LANES)
                        xx1 = jnp.maximum(x1_v[sl], xi1)
                        yy1 = jnp.maximum(y1_v[sl], yi1)
                        xx2 = jnp.minimum(x2_v[sl], xi2)
                        yy2 = jnp.minimum(y2_v[sl], yi2)
                        inter = (jnp.maximum(xx2 - xx1, 0.0)
                                 * jnp.maximum(yy2 - yy1, 0.0))
                        union = (ai + area_v[sl]) - inter
                        iou = inter / jnp.maximum(union, 1e-9)
                        idx = lax.iota(jnp.int32, _LANES) + base
                        sup = (iou > _NMS_T) & (idx > i)
                        keep_v[sl] = jnp.where(sup, 0.0, keep_v[sl])

                return carry

            lax.fori_loop(0, n_real, outer, 0)

            @plsc.parallel_loop(0, nblk, unroll=4)
            def _mask_blk(b):
                sl = pl.ds(b * _LANES, _LANES)
                k = keep_v[sl] > 0.0
                x1_v[sl] = jnp.where(k, x1_v[sl], 0.0)
                y1_v[sl] = jnp.where(k, y1_v[sl], 0.0)
                x2_v[sl] = jnp.where(k, x2_v[sl], 0.0)
                y2_v[sl] = jnp.where(k, y2_v[sl], 0.0)
                s_v[sl] = jnp.where(k, s_v[sl], 0.0)

            pltpu.sync_copy(x1_v, ox1_h.at[wid])
            pltpu.sync_copy(y1_v, oy1_h.at[wid])
            pltpu.sync_copy(x2_v, ox2_h.at[wid])
            pltpu.sync_copy(y2_v, oy2_h.at[wid])
            pltpu.sync_copy(s_v, os_h.at[wid])
            pltpu.sync_copy(keep_v, okeep_h.at[wid])

    mesh = plsc.VectorSubcoreMesh(core_axis_name="c", subcore_axis_name="s")
    out_t = [jax.ShapeDtypeStruct((n_cls, n_pad), jnp.float32)] * 6
    scratch = [pltpu.VMEM((n_pad,), jnp.float32) for _ in range(7)]
    return pl.kernel(body, out_type=out_t, mesh=mesh, scratch_types=scratch)


def kernel(raw_cls_bbox, raw_prob):
    n, ncls = raw_prob.shape
    L = ncls - 1
    n_pad = -(-n // _STEP) * _STEP

    cls_bbox = raw_cls_bbox.reshape(n, ncls, 4)
    boxes_pc = jnp.transpose(cls_bbox[:, 1:, :], (1, 0, 2))
    probs_pc = jnp.transpose(raw_prob[:, 1:], (1, 0))
    scores = jnp.where(probs_pc > _SCORE_T, probs_pc, -jnp.inf)
    order = jnp.argsort(-scores, axis=1)
    b = jnp.take_along_axis(boxes_pc, order[:, :, None], axis=1)
    s = jnp.take_along_axis(scores, order, axis=1)

    pad = n_pad - n
    x1 = jnp.pad(b[:, :, 0], ((0, 0), (0, pad)))
    y1 = jnp.pad(b[:, :, 1], ((0, 0), (0, pad)))
    x2 = jnp.pad(b[:, :, 2], ((0, 0), (0, pad)))
    y2 = jnp.pad(b[:, :, 3], ((0, 0), (0, pad)))
    sp = jnp.pad(s, ((0, 0), (0, pad)), constant_values=-jnp.inf)

    fn = _make_nms_kernel(L, n_pad, n)
    ox1, oy1, ox2, oy2, os_, okeep = fn(x1, y1, x2, y2, sp)

    bbox = jnp.stack(
        [ox1[:, :n], oy1[:, :n], ox2[:, :n], oy2[:, :n]], axis=-1
    ).reshape(L * n, 4)
    score = os_[:, :n].reshape(L * n)
    keep = (okeep[:, :n] > 0.0).reshape(L * n)
    label = jnp.repeat(jnp.arange(L, dtype=jnp.int32), n)
    return bbox, label, score, keep


# speedup vs baseline: 47.0131x; 1.0066x over previous
import jax
import jax.numpy as jnp
from jax import lax
from jax.experimental import pallas as pl
from jax.experimental.pallas import tpu as pltpu
from jax.experimental.pallas import tpu_sc as plsc

_NMS_T = 0.3
_SCORE_T = 0.05
_LANES = 16


def _make_nms_kernel(n_cls, n_pad, n_real):
    nblk = n_pad // _LANES

    def body(x1x_h, y1x_h, x2x_h, y2x_h, sx_h, p2x_h, st_h, en_h,
             x1s_h, y1s_h, x2s_h, y2s_h, sp_h,
             ox1_h, oy1_h, ox2_h, oy2_h, os_h, okeep_h,
             vx1, vy1, vx2, vy2, varea, vsx, vkeep, vp2x, vst, ven,
             sx1, sy1, sx2, sy2, ssp, skeep):
        wid = lax.axis_index("s") * 2 + lax.axis_index("c")

        @pl.when(wid < n_cls)
        def _work():
            pltpu.sync_copy(x1x_h.at[wid], vx1)
            pltpu.sync_copy(y1x_h.at[wid], vy1)
            pltpu.sync_copy(x2x_h.at[wid], vx2)
            pltpu.sync_copy(y2x_h.at[wid], vy2)
            pltpu.sync_copy(sx_h.at[wid], vsx)
            pltpu.sync_copy(p2x_h.at[wid], vp2x)
            pltpu.sync_copy(st_h.at[wid], vst)
            pltpu.sync_copy(en_h.at[wid], ven)
            pltpu.sync_copy(x1s_h.at[wid], sx1)
            pltpu.sync_copy(y1s_h.at[wid], sy1)
            pltpu.sync_copy(x2s_h.at[wid], sx2)
            pltpu.sync_copy(y2s_h.at[wid], sy2)
            pltpu.sync_copy(sp_h.at[wid], ssp)

            @plsc.parallel_loop(0, nblk, unroll=4)
            def _init_blk(bq):
                sl = pl.ds(bq * _LANES, _LANES)
                w = jnp.maximum(vx2[sl] - vx1[sl], 0.0)
                h = jnp.maximum(vy2[sl] - vy1[sl], 0.0)
                varea[sl] = w * h
                vkeep[sl] = jnp.where(vsx[sl] > -jnp.inf, 1.0, 0.0)

            def outer(i, carry):
                sli = pl.ds(i, _LANES)
                pxi = vp2x[sli][0]
                ki = vkeep[pl.ds(pxi, _LANES)][0]

                @pl.when(ki > 0.0)
                def _suppress():
                    xi1 = sx1[sli][0]
                    yi1 = sy1[sli][0]
                    xi2 = sx2[sli][0]
                    yi2 = sy2[sli][0]
                    ai = (jnp.maximum(xi2 - xi1, 0.0)
                          * jnp.maximum(yi2 - yi1, 0.0))
                    g0 = vst[sli][0] // _LANES
                    g1 = (ven[sli][0] + (_LANES - 1)) // _LANES

                    @plsc.parallel_loop(g0, g1, unroll=4)
                    def _sweep(g):
                        base = g * _LANES
                        sl = pl.ds(base, _LANES)
                        xx1 = jnp.maximum(vx1[sl], xi1)
                        yy1 = jnp.maximum(vy1[sl], yi1)
                        xx2 = jnp.minimum(vx2[sl], xi2)
                        yy2 = jnp.minimum(vy2[sl], yi2)
                        inter = (jnp.maximum(xx2 - xx1, 0.0)
                                 * jnp.maximum(yy2 - yy1, 0.0))
                        union = (ai + varea[sl]) - inter
                        iou = inter / jnp.maximum(union, 1e-9)
                        xidx = lax.iota(jnp.int32, _LANES) + base
                        sup = (iou > _NMS_T) & (xidx != pxi)
                        vkeep[sl] = jnp.where(sup, 0.0, vkeep[sl])

                return carry

            lax.fori_loop(0, n_real, outer, 0)

            lane = lax.iota(jnp.int32, _LANES)

            @plsc.parallel_loop(0, nblk, unroll=2)
            def _mask_blk(bq):
                sl = pl.ds(bq * _LANES, _LANES)
                pvec = vp2x[sl]
                kx = jnp.zeros((_LANES,), jnp.float32)
                for q in range(_LANES):
                    kq = vkeep[pl.ds(pvec[q], _LANES)][0]
                    kx = jnp.where(lane == q, kq, kx)
                km = kx > 0.0
                sx1[sl] = jnp.where(km, sx1[sl], 0.0)
                sy1[sl] = jnp.where(km, sy1[sl], 0.0)
                sx2[sl] = jnp.where(km, sx2[sl], 0.0)
                sy2[sl] = jnp.where(km, sy2[sl], 0.0)
                ssp[sl] = jnp.where(km, ssp[sl], 0.0)
                skeep[sl] = jnp.where(km, 1.0, 0.0)

            pltpu.sync_copy(sx1, ox1_h.at[wid])
            pltpu.sync_copy(sy1, oy1_h.at[wid])
            pltpu.sync_copy(sx2, ox2_h.at[wid])
            pltpu.sync_copy(sy2, oy2_h.at[wid])
            pltpu.sync_copy(ssp, os_h.at[wid])
            pltpu.sync_copy(skeep, okeep_h.at[wid])

    mesh = plsc.VectorSubcoreMesh(core_axis_name="c", subcore_axis_name="s")
    out_t = [jax.ShapeDtypeStruct((n_cls, n_pad), jnp.float32)] * 6
    scratch = (
        [pltpu.VMEM((n_pad,), jnp.float32) for _ in range(6)]
        + [pltpu.VMEM((n_pad + _LANES,), jnp.float32)]
        + [pltpu.VMEM((n_pad,), jnp.int32) for _ in range(3)]
        + [pltpu.VMEM((n_pad,), jnp.float32) for _ in range(6)]
    )
    return pl.kernel(body, out_type=out_t, mesh=mesh, scratch_types=scratch)


def kernel(raw_cls_bbox, raw_prob):
    n, ncls = raw_prob.shape
    L = ncls - 1
    n_pad = -(-n // 64) * 64

    cls_bbox = raw_cls_bbox.reshape(n, ncls, 4)
    boxes_pc = jnp.transpose(cls_bbox[:, 1:, :], (1, 0, 2))
    probs_pc = jnp.transpose(raw_prob[:, 1:], (1, 0))
    scores = jnp.where(probs_pc > _SCORE_T, probs_pc, -jnp.inf)
    order = jnp.argsort(-scores, axis=1)
    b = jnp.take_along_axis(boxes_pc, order[:, :, None], axis=1)
    s = jnp.take_along_axis(scores, order, axis=1)

    pad = n_pad - n
    x1s = jnp.pad(b[:, :, 0], ((0, 0), (0, pad)))
    y1s = jnp.pad(b[:, :, 1], ((0, 0), (0, pad)))
    x2s = jnp.pad(b[:, :, 2], ((0, 0), (0, pad)))
    y2s = jnp.pad(b[:, :, 3], ((0, 0), (0, pad)))
    sp = jnp.pad(s, ((0, 0), (0, pad)), constant_values=-jnp.inf)

    xord = jnp.argsort(x1s, axis=1)
    p2x = jnp.argsort(xord, axis=1).astype(jnp.int32)
    x1x = jnp.take_along_axis(x1s, xord, axis=1)
    y1x = jnp.take_along_axis(y1s, xord, axis=1)
    x2x = jnp.take_along_axis(x2s, xord, axis=1)
    y2x = jnp.take_along_axis(y2s, xord, axis=1)
    sx = jnp.take_along_axis(sp, xord, axis=1)

    maxw = jnp.max(x2x - x1x, axis=1, keepdims=True)
    lo = x1s - maxw - 1.0
    vss = jax.vmap(jnp.searchsorted, in_axes=(0, 0))
    start = vss(x1x, lo).astype(jnp.int32)
    end = vss(x1x, x2s).astype(jnp.int32)

    fn = _make_nms_kernel(L, n_pad, n)
    ox1, oy1, ox2, oy2, os_, okeep = fn(
        x1x, y1x, x2x, y2x, sx, p2x, start, end, x1s, y1s, x2s, y2s, sp)

    bbox = jnp.stack(
        [ox1[:, :n], oy1[:, :n], ox2[:, :n], oy2[:, :n]], axis=-1
    ).reshape(L * n, 4)
    score = os_[:, :n].reshape(L * n)
    keep = (okeep[:, :n] > 0.0).reshape(L * n)
    label = jnp.repeat(jnp.arange(L, dtype=jnp.int32), n)
    return bbox, label, score, keep
